# baseline (speedup 1.0000x reference)
import functools

import jax
import jax.numpy as jnp
from jax import lax
from jax.experimental import pallas as pl
from jax.experimental.pallas import tpu as pltpu
from jax.experimental.pallas import tpu_sc as plsc


def _scan_body(bias_ref, hid_ref, w_ref, scores_ref):
    part = lax.dot_general(
        w_ref[...], hid_ref[...],
        dimension_numbers=(((1,), (1,)), ((), ())),
        preferred_element_type=jnp.float32,
    )
    scores_ref[0] = part + bias_ref[0]


def _select_body(k_ref, scores_ref, mask_ref, *, idx_bits):
    s = scores_ref[...]
    B = s.shape[0]
    u = lax.bitcast_convert_type(s, jnp.uint32)
    neg = u >= jnp.uint32(0x80000000)
    key = jnp.where(neg, ~u, u | jnp.uint32(0x80000000))
    kk = k_ref[...]

    th = jnp.zeros((B, 1), jnp.uint32)
    for shift in range(30, -2, -2):
        d = jnp.zeros((B, 1), jnp.int32)
        for c in (1, 2, 3):
            cnt = jnp.sum((key >= (th | jnp.uint32(c << shift)))
                          .astype(jnp.int32), axis=1, keepdims=True)
            d = d + (cnt >= kk).astype(jnp.int32)
        th = th | (d.astype(jnp.uint32) << shift)

    gt = key > th
    tie = key == th
    need = kk - jnp.sum(gt.astype(jnp.int32), axis=1, keepdims=True)
    idxs = lax.broadcasted_iota(jnp.int32, s.shape, 1)

    rsel = jnp.zeros((B, 1), jnp.int32)
    for shift in range(idx_bits - 2, -2, -2):
        d = jnp.zeros((B, 1), jnp.int32)
        for c in (1, 2, 3):
            cnt = jnp.sum((tie & (idxs < (rsel + jnp.int32(c << shift))))
                          .astype(jnp.int32), axis=1, keepdims=True)
            d = d + (cnt < need).astype(jnp.int32)
        rsel = rsel + (d << shift)

    mask_ref[...] = (gt | (tie & (idxs <= rsel))).astype(jnp.int32)


def _sc_select_body(scores_hbm, k_hbm, out_hbm, row_v, k_v, hist_v, mask_v):
    B, T = scores_hbm.shape
    NV = T // 16

    wid = lax.axis_index("s") * 2 + lax.axis_index("c")

    @pl.when(wid < B)
    def _():
        pltpu.sync_copy(scores_hbm.at[wid], row_v)
        pltpu.sync_copy(k_hbm, k_v)
        kvec = k_v[...]
        lane = lax.iota(jnp.int32, 16)
        k_row = lax.reduce_sum(jnp.where(lane == wid, kvec, 0), (0,))

        ones = jnp.ones((16,), jnp.int32)
        zeros16 = jnp.zeros((16,), jnp.int32)

        def key_at(i):
            f = row_v[pl.ds(i * 16, 16)]
            u = plsc.bitcast(f, jnp.uint32)
            neg = u >= jnp.uint32(0x80000000)
            return jnp.where(neg, ~u, u | jnp.uint32(0x80000000))

        def level(l, prefix, k_rem):
            shift_pref = 32 - 8 * l
            shift_bkt = 24 - 8 * l

            def z(i, _):
                hist_v[pl.ds(i * 16, 16)] = zeros16
                return 0
            lax.fori_loop(0, 16, z, 0)

            def h(i, _):
                key = key_at(i)
                if l > 0:
                    elig = (key >> jnp.uint32(shift_pref)) == prefix
                else:
                    elig = key == key
                bkt = ((key >> jnp.uint32(shift_bkt))
                       & jnp.uint32(0xFF)).astype(jnp.int32)
                plsc.addupdate_scatter(hist_v, [bkt], ones, mask=elig)
                return 0
            lax.fori_loop(0, NV, h, 0)

            def s(j, c):
                acc, tb, g, done = c
                jj = 15 - j
                hv = hist_v[pl.ds(jj * 16, 16)]
                csum = plsc.cumsum(hv)
                vsum = lax.reduce_sum(hv, (0,))
                suff = vsum - csum + hv
                cond = (acc + suff) >= k_rem
                npc = lax.reduce_sum(cond.astype(jnp.int32), (0,))
                hit = jnp.logical_and(jnp.logical_not(done), npc > 0)
                above_in_vreg = lax.reduce_sum(
                    jnp.where(lane == npc - 1, suff - hv, 0), (0,))
                tb = jnp.where(hit, jj * 16 + npc - 1, tb)
                g = jnp.where(hit, acc + above_in_vreg, g)
                done = jnp.logical_or(done, npc > 0)
                return (acc + vsum, tb, g, done)
            _, tb, g, _ = lax.fori_loop(
                0, 16, s,
                (jnp.int32(0), jnp.int32(0), jnp.int32(0), False))

            prefix = (prefix << jnp.uint32(8)) | tb.astype(jnp.uint32)
            return prefix, k_rem - g

        prefix, need = level(0, jnp.uint32(0), k_row)
        prefix, need = level(1, prefix, need)
        prefix, need = level(2, prefix, need)
        th, need = level(3, prefix, need)

        def m(i, carry):
            key = key_at(i)
            gt = key > th
            tie = key == th
            tcs = plsc.cumsum(tie.astype(jnp.int32))
            rank = carry + tcs
            sel = jnp.logical_or(gt, jnp.logical_and(tie, rank <= need))
            mask_v[pl.ds(i * 16, 16)] = sel.astype(jnp.int32)
            return carry + lax.reduce_sum(tie.astype(jnp.int32), (0,))
        lax.fori_loop(0, NV, m, jnp.int32(0))

        pltpu.sync_copy(mask_v, out_hbm.at[wid])


def _sc_select(scores, k):
    B, T = scores.shape
    k_pad = jnp.zeros((16,), jnp.int32).at[:B].set(k)
    mesh = plsc.VectorSubcoreMesh(core_axis_name="c", subcore_axis_name="s")
    f = pl.kernel(
        _sc_select_body,
        mesh=mesh,
        out_type=jax.ShapeDtypeStruct((B, T), jnp.int32),
        scratch_types=[
            pltpu.VMEM((T,), jnp.float32),
            pltpu.VMEM((16,), jnp.int32),
            pltpu.VMEM((256,), jnp.int32),
            pltpu.VMEM((T,), jnp.int32),
        ],
        compiler_params=pltpu.CompilerParams(needs_layout_passes=False),
    )
    return f(scores, k_pad)


def kernel(hidden, keep_ratio, W, b):
    B, T, C = hidden.shape
    T_BLK = 512
    N = (B * T) // T_BLK
    idx_bits = (T - 1).bit_length()
    if idx_bits % 2:
        idx_bits += 1

    kr = jnp.clip(keep_ratio, 0.1, 1.0)
    k = jnp.maximum(1, (kr * T).astype(jnp.int32))
    w_row = W.reshape(1, C)
    hid2d = hidden.reshape(B * T, C)

    scores3 = pl.pallas_call(
        _scan_body,
        grid=(N,),
        in_specs=[
            pl.BlockSpec(memory_space=pltpu.SMEM),
            pl.BlockSpec((T_BLK, C), lambda i: (i, 0)),
            pl.BlockSpec((1, C), lambda i: (0, 0)),
        ],
        out_specs=pl.BlockSpec((1, 1, T_BLK), lambda i: (i, 0, 0)),
        out_shape=jax.ShapeDtypeStruct((N, 1, T_BLK), jnp.float32),
        compiler_params=pltpu.CompilerParams(
            dimension_semantics=("arbitrary",),
        ),
    )(b, hid2d, w_row)
    scores = scores3.reshape(B, T)

    mask_i32 = _sc_select(scores, k)

    return (mask_i32.astype(jnp.bool_), scores)


# speedup vs baseline: 1.2537x; 1.2537x over previous
import functools

import jax
import jax.numpy as jnp
from jax import lax
from jax.experimental import pallas as pl
from jax.experimental.pallas import tpu as pltpu


def _scan_body(bias_ref, hid_ref, w_ref, scores_ref):
    part = lax.dot_general(
        w_ref[...], hid_ref[...],
        dimension_numbers=(((1,), (1,)), ((), ())),
        preferred_element_type=jnp.float32,
    )
    scores_ref[0] = part + bias_ref[0]


def _select_body(k_ref, scores_ref, mask_ref, *, idx_bits):
    s = scores_ref[...]
    B = s.shape[0]
    u = lax.bitcast_convert_type(s, jnp.uint32)
    neg = u >= jnp.uint32(0x80000000)
    key = jnp.where(neg, ~u, u | jnp.uint32(0x80000000))
    kk = k_ref[...]

    th = jnp.zeros((B, 1), jnp.uint32)
    for shift in range(30, -2, -2):
        d = jnp.zeros((B, 1), jnp.int32)
        for c in (1, 2, 3):
            cnt = jnp.sum((key >= (th | jnp.uint32(c << shift)))
                          .astype(jnp.int32), axis=1, keepdims=True)
            d = d + (cnt >= kk).astype(jnp.int32)
        th = th | (d.astype(jnp.uint32) << shift)

    gt = key > th
    tie = key == th
    need = kk - jnp.sum(gt.astype(jnp.int32), axis=1, keepdims=True)
    idxs = lax.broadcasted_iota(jnp.int32, s.shape, 1)

    rsel = jnp.zeros((B, 1), jnp.int32)
    for shift in range(idx_bits - 2, -2, -2):
        d = jnp.zeros((B, 1), jnp.int32)
        for c in (1, 2, 3):
            cnt = jnp.sum((tie & (idxs < (rsel + jnp.int32(c << shift))))
                          .astype(jnp.int32), axis=1, keepdims=True)
            d = d + (cnt < need).astype(jnp.int32)
        rsel = rsel + (d << shift)

    mask_ref[...] = (gt | (tie & (idxs <= rsel))).astype(jnp.int32)


def kernel(hidden, keep_ratio, W, b):
    B, T, C = hidden.shape
    T_BLK = 512
    N = (B * T) // T_BLK
    idx_bits = (T - 1).bit_length()
    if idx_bits % 2:
        idx_bits += 1

    kr = jnp.clip(keep_ratio, 0.1, 1.0)
    k = jnp.maximum(1, (kr * T).astype(jnp.int32))
    w_row = W.reshape(1, C)
    hid2d = hidden.reshape(B * T, C)

    scores3 = pl.pallas_call(
        _scan_body,
        grid=(N,),
        in_specs=[
            pl.BlockSpec(memory_space=pltpu.SMEM),
            pl.BlockSpec((T_BLK, C), lambda i: (i, 0)),
            pl.BlockSpec((1, C), lambda i: (0, 0)),
        ],
        out_specs=pl.BlockSpec((1, 1, T_BLK), lambda i: (i, 0, 0)),
        out_shape=jax.ShapeDtypeStruct((N, 1, T_BLK), jnp.float32),
        compiler_params=pltpu.CompilerParams(
            dimension_semantics=("arbitrary",),
        ),
    )(b, hid2d, w_row)
    scores = scores3.reshape(B, T)

    mask_i32 = pl.pallas_call(
        functools.partial(_select_body, idx_bits=idx_bits),
        in_specs=[
            pl.BlockSpec((B, 1), lambda: (0, 0)),
            pl.BlockSpec((B, T), lambda: (0, 0)),
        ],
        out_specs=pl.BlockSpec((B, T), lambda: (0, 0)),
        out_shape=jax.ShapeDtypeStruct((B, T), jnp.int32),
    )(k.reshape(B, 1), scores)

    return (mask_i32.astype(jnp.bool_), scores)
